# baseline (device time: 13167 ns/iter reference)
import jax
import jax.numpy as jnp
from jax import lax
from jax.experimental import pallas as pl
from jax.experimental.pallas import tpu as pltpu

N_DEV = 16
B = 2
S = 256
HALO = 128
SKV = S + 2 * HALO
HQ = 4
DH = 64
F = HQ * DH
D_MODEL = 512
SCALE = 0.125
NEG = -1e9
RB = 128
W = RB + 2 * HALO

LEFT, RIGHT = 0, 1


def kernel(x, Wq, K_ext, V_ext, Wo):
    K2 = K_ext.reshape(B, S, F)
    V2 = V_ext.reshape(B, S, F)

    def body(x_ref, wq_ref, k_ref, v_ref, wo_ref, out_ref,
             kv_buf, stage, send_sems, recv_sems):
        my = lax.axis_index("i")
        has_left = my > 0
        has_right = my < N_DEV - 1

        k_local = k_ref[...].astype(jnp.bfloat16)
        v_local = v_ref[...].astype(jnp.bfloat16)
        kv_buf[:, HALO:HALO + S, :F] = k_local
        kv_buf[:, HALO:HALO + S, F:] = v_local
        stage[0, :, :, :F] = k_local[:, :HALO, :]
        stage[0, :, :, F:] = v_local[:, :HALO, :]
        stage[1, :, :, :F] = k_local[:, S - HALO:, :]
        stage[1, :, :, F:] = v_local[:, S - HALO:, :]

        @pl.when(jnp.logical_not(has_left))
        def _():
            kv_buf[:, :HALO, :] = jnp.zeros((B, HALO, 2 * F), jnp.bfloat16)

        @pl.when(jnp.logical_not(has_right))
        def _():
            kv_buf[:, HALO + S:, :] = jnp.zeros((B, HALO, 2 * F),
                                                jnp.bfloat16)

        barrier = pltpu.get_barrier_semaphore()

        @pl.when(has_left)
        def _():
            pl.semaphore_signal(barrier, inc=1, device_id=(my - 1,),
                                device_id_type=pl.DeviceIdType.MESH)

        @pl.when(has_right)
        def _():
            pl.semaphore_signal(barrier, inc=1, device_id=(my + 1,),
                                device_id_type=pl.DeviceIdType.MESH)

        @pl.when(has_left)
        def _():
            pl.semaphore_wait(barrier, 1)

        @pl.when(has_right)
        def _():
            pl.semaphore_wait(barrier, 1)

        def halo_rdma(slot, region_start, sem_idx, target):
            return pltpu.make_async_remote_copy(
                src_ref=stage.at[slot],
                dst_ref=kv_buf.at[:, pl.ds(region_start, HALO), :],
                send_sem=send_sems.at[sem_idx],
                recv_sem=recv_sems.at[sem_idx],
                device_id=(target,),
                device_id_type=pl.DeviceIdType.MESH,
            )

        @pl.when(has_left)
        def _():
            halo_rdma(0, HALO + S, RIGHT, my - 1).start()

        @pl.when(has_right)
        def _():
            halo_rdma(1, 0, LEFT, my + 1).start()

        wq = wq_ref[...].astype(jnp.bfloat16)
        qs = []
        for b in range(B):
            q = lax.dot_general(x_ref[b].astype(jnp.bfloat16), wq,
                                (((1,), (0,)), ((), ())),
                                preferred_element_type=jnp.float32)
            qs.append((q * SCALE).astype(jnp.bfloat16))

        def wait_region(region_start, sem_idx):
            pltpu.make_async_remote_copy(
                src_ref=stage.at[0],
                dst_ref=kv_buf.at[:, pl.ds(region_start, HALO), :],
                send_sem=send_sems.at[sem_idx],
                recv_sem=recv_sems.at[sem_idx],
                device_id=(my,),
                device_id_type=pl.DeviceIdType.MESH,
            ).wait_recv()

        def attend_block(blk):
            r = lax.broadcasted_iota(jnp.int32, (RB, W), 0)
            c = lax.broadcasted_iota(jnp.int32, (RB, W), 1)
            kg = my * S - HALO + blk * RB + c
            valid = ((c >= r) & (c <= r + 2 * HALO)
                     & (kg >= 0) & (kg < N_DEV * S))
            per_b = []
            for b in range(B):
                kvb = kv_buf[b, blk * RB:blk * RB + W, :]
                ctxs = []
                for h in range(HQ):
                    qh = qs[b][blk * RB:(blk + 1) * RB,
                               h * DH:(h + 1) * DH]
                    s = lax.dot_general(qh, kvb[:, h * DH:(h + 1) * DH],
                                        (((1,), (1,)), ((), ())),
                                        preferred_element_type=jnp.float32)
                    w = jnp.exp(jnp.where(valid, s, NEG))
                    denom = jnp.sum(w, axis=-1, keepdims=True)
                    ctx = lax.dot_general(
                        w.astype(jnp.bfloat16),
                        kvb[:, F + h * DH:F + (h + 1) * DH],
                        (((1,), (0,)), ((), ())),
                        preferred_element_type=jnp.float32)
                    ctxs.append(ctx / denom)
                per_b.append(jnp.concatenate(ctxs, axis=1))
            return per_b

        @pl.when(has_left)
        def _():
            wait_region(0, LEFT)

        blk0 = attend_block(0)

        @pl.when(has_right)
        def _():
            wait_region(HALO + S, RIGHT)

        blk1 = attend_block(1)

        wo = wo_ref[...].astype(jnp.bfloat16)
        for b in range(B):
            ctx_b = jnp.concatenate([blk0[b], blk1[b]],
                                    axis=0).astype(jnp.bfloat16)
            out_ref[b] = lax.dot_general(ctx_b, wo, (((1,), (0,)), ((), ())),
                                         preferred_element_type=jnp.float32)

        @pl.when(has_left)
        def _():
            halo_rdma(0, HALO + S, RIGHT, my - 1).wait_send()

        @pl.when(has_right)
        def _():
            halo_rdma(1, 0, LEFT, my + 1).wait_send()

    return pl.pallas_call(
        body,
        out_shape=jax.ShapeDtypeStruct((B, S, D_MODEL), jnp.float32),
        in_specs=[pl.BlockSpec(memory_space=pltpu.VMEM)] * 5,
        out_specs=pl.BlockSpec(memory_space=pltpu.VMEM),
        scratch_shapes=[
            pltpu.VMEM((B, SKV, 2 * F), jnp.bfloat16),
            pltpu.VMEM((2, B, HALO, 2 * F), jnp.bfloat16),
            pltpu.SemaphoreType.DMA((2,)),
            pltpu.SemaphoreType.DMA((2,)),
        ],
        compiler_params=pltpu.CompilerParams(collective_id=0),
    )(x, Wq, K2, V2, Wo)


# device time: 13125 ns/iter; 1.0032x vs baseline; 1.0032x over previous
import jax
import jax.numpy as jnp
from jax import lax
from jax.experimental import pallas as pl
from jax.experimental.pallas import tpu as pltpu

N_DEV = 16
B = 2
S = 256
HALO = 128
SKV = S + 2 * HALO
HQ = 4
DH = 64
F = HQ * DH
D_MODEL = 512
SCALE = 0.125
NEG = -1e9
RB = 128
W = RB + 2 * HALO

LEFT, RIGHT = 0, 1


def kernel(x, Wq, K_ext, V_ext, Wo):
    K2 = K_ext.reshape(B, S, F)
    V2 = V_ext.reshape(B, S, F)

    def body(x_ref, wq_ref, k_ref, v_ref, wo_ref, out_ref,
             kv_buf, send_sems, recv_sems):
        my = lax.axis_index("i")
        has_left = my > 0
        has_right = my < N_DEV - 1

        k_local = k_ref[...].astype(jnp.bfloat16)
        v_local = v_ref[...].astype(jnp.bfloat16)
        kv_buf[:, HALO:HALO + S, :F] = k_local
        kv_buf[:, HALO:HALO + S, F:] = v_local

        @pl.when(jnp.logical_not(has_left))
        def _():
            kv_buf[:, :HALO, :] = jnp.zeros((B, HALO, 2 * F), jnp.bfloat16)

        @pl.when(jnp.logical_not(has_right))
        def _():
            kv_buf[:, HALO + S:, :] = jnp.zeros((B, HALO, 2 * F),
                                                jnp.bfloat16)

        barrier = pltpu.get_barrier_semaphore()

        @pl.when(has_left)
        def _():
            pl.semaphore_signal(barrier, inc=1, device_id=(my - 1,),
                                device_id_type=pl.DeviceIdType.MESH)

        @pl.when(has_right)
        def _():
            pl.semaphore_signal(barrier, inc=1, device_id=(my + 1,),
                                device_id_type=pl.DeviceIdType.MESH)

        @pl.when(has_left)
        def _():
            pl.semaphore_wait(barrier, 1)

        @pl.when(has_right)
        def _():
            pl.semaphore_wait(barrier, 1)

        def halo_rdma(src_start, region_start, sem_idx, target):
            return pltpu.make_async_remote_copy(
                src_ref=kv_buf.at[:, pl.ds(src_start, HALO), :],
                dst_ref=kv_buf.at[:, pl.ds(region_start, HALO), :],
                send_sem=send_sems.at[sem_idx],
                recv_sem=recv_sems.at[sem_idx],
                device_id=(target,),
                device_id_type=pl.DeviceIdType.MESH,
            )

        @pl.when(has_left)
        def _():
            halo_rdma(HALO, HALO + S, RIGHT, my - 1).start()

        @pl.when(has_right)
        def _():
            halo_rdma(S, 0, LEFT, my + 1).start()

        wq = wq_ref[...].astype(jnp.bfloat16)
        qs = []
        for b in range(B):
            q = lax.dot_general(x_ref[b].astype(jnp.bfloat16), wq,
                                (((1,), (0,)), ((), ())),
                                preferred_element_type=jnp.float32)
            qs.append((q * SCALE).astype(jnp.bfloat16))

        def wait_region(region_start, sem_idx):
            pltpu.make_async_remote_copy(
                src_ref=kv_buf.at[:, pl.ds(0, HALO), :],
                dst_ref=kv_buf.at[:, pl.ds(region_start, HALO), :],
                send_sem=send_sems.at[sem_idx],
                recv_sem=recv_sems.at[sem_idx],
                device_id=(my,),
                device_id_type=pl.DeviceIdType.MESH,
            ).wait_recv()

        def attend_block(blk):
            r = lax.broadcasted_iota(jnp.int32, (RB, W), 0)
            c = lax.broadcasted_iota(jnp.int32, (RB, W), 1)
            kg = my * S - HALO + blk * RB + c
            valid = ((c >= r) & (c <= r + 2 * HALO)
                     & (kg >= 0) & (kg < N_DEV * S))
            per_b = []
            for b in range(B):
                kvb = kv_buf[b, blk * RB:blk * RB + W, :]
                ctxs = []
                for h in range(HQ):
                    qh = qs[b][blk * RB:(blk + 1) * RB,
                               h * DH:(h + 1) * DH]
                    s = lax.dot_general(qh, kvb[:, h * DH:(h + 1) * DH],
                                        (((1,), (1,)), ((), ())),
                                        preferred_element_type=jnp.float32)
                    w = jnp.exp(jnp.where(valid, s, NEG))
                    denom = jnp.sum(w, axis=-1, keepdims=True)
                    ctx = lax.dot_general(
                        w.astype(jnp.bfloat16),
                        kvb[:, F + h * DH:F + (h + 1) * DH],
                        (((1,), (0,)), ((), ())),
                        preferred_element_type=jnp.float32)
                    ctxs.append(ctx / denom)
                per_b.append(jnp.concatenate(ctxs, axis=1))
            return per_b

        wo = wo_ref[...].astype(jnp.bfloat16)

        @pl.when(has_left)
        def _():
            wait_region(0, LEFT)

        blk0 = attend_block(0)
        for b in range(B):
            out_ref[b, :RB] = lax.dot_general(
                blk0[b].astype(jnp.bfloat16), wo, (((1,), (0,)), ((), ())),
                preferred_element_type=jnp.float32)

        @pl.when(has_right)
        def _():
            wait_region(HALO + S, RIGHT)

        blk1 = attend_block(1)
        for b in range(B):
            out_ref[b, RB:] = lax.dot_general(
                blk1[b].astype(jnp.bfloat16), wo, (((1,), (0,)), ((), ())),
                preferred_element_type=jnp.float32)

        @pl.when(has_left)
        def _():
            halo_rdma(HALO, HALO + S, RIGHT, my - 1).wait_send()

        @pl.when(has_right)
        def _():
            halo_rdma(S, 0, LEFT, my + 1).wait_send()

    return pl.pallas_call(
        body,
        out_shape=jax.ShapeDtypeStruct((B, S, D_MODEL), jnp.float32),
        in_specs=[pl.BlockSpec(memory_space=pltpu.VMEM)] * 5,
        out_specs=pl.BlockSpec(memory_space=pltpu.VMEM),
        scratch_shapes=[
            pltpu.VMEM((B, SKV, 2 * F), jnp.bfloat16),
            pltpu.SemaphoreType.DMA((2,)),
            pltpu.SemaphoreType.DMA((2,)),
        ],
        compiler_params=pltpu.CompilerParams(collective_id=0),
    )(x, Wq, K2, V2, Wo)
